# baseline (speedup 1.0000x reference)
import functools

import jax
import jax.numpy as jnp
from jax import lax
from jax.experimental import pallas as pl
from jax.experimental.pallas import tpu as pltpu
from jax.experimental.pallas import tpu_sc as plsc

VOCAB = 100000
EMBED = 32
HIDDEN = 64
NTOK = 256

_VTILE = 4096


@functools.lru_cache(maxsize=1)
def _make_sc_gather():
    info = plsc.get_sparse_core_info()
    nc, ns = info.num_cores, info.num_subcores
    nw = nc * ns
    b_per_w = NTOK // nw
    mesh = plsc.VectorSubcoreMesh(core_axis_name="c", subcore_axis_name="s")

    @functools.partial(
        pl.kernel,
        mesh=mesh,
        out_type=jax.ShapeDtypeStruct((NTOK, EMBED), jnp.float32),
        scratch_types=[
            pltpu.VMEM((b_per_w,), jnp.int32),
            pltpu.VMEM((b_per_w, EMBED), jnp.float32),
            pltpu.SemaphoreType.DMA,
        ],
        compiler_params=pltpu.CompilerParams(use_tc_tiling_on_sc=False),
    )
    def gather_kernel(table_hbm, idx_hbm, out_hbm, idx_v, rows_v, sem):
        wid = lax.axis_index("s") * nc + lax.axis_index("c")
        base = wid * b_per_w
        pltpu.sync_copy(idx_hbm.at[pl.ds(base, b_per_w)], idx_v)
        pltpu.async_copy(table_hbm.at[idx_v], rows_v, sem).wait()
        pltpu.sync_copy(rows_v, out_hbm.at[pl.ds(base, b_per_w)])

    return gather_kernel


def _mlp_body(x_ref, w1_ref, b1_ref, wh_ref, out_ref):
    h = lax.dot_general(
        x_ref[...], w1_ref[...],
        (((1,), (1,)), ((), ())),
        preferred_element_type=jnp.float32,
    ) + b1_ref[...]
    out_ref[...] = lax.dot_general(
        h, wh_ref[...],
        (((1,), (1,)), ((), ())),
        preferred_element_type=jnp.float32,
    )


def _mlp_tc(x, w1, b1_2d, wh, interpret=False):
    grid = (pl.cdiv(VOCAB, _VTILE),)
    return pl.pallas_call(
        _mlp_body,
        grid=grid,
        in_specs=[
            pl.BlockSpec((NTOK, EMBED), lambda i: (0, 0)),
            pl.BlockSpec((HIDDEN, EMBED), lambda i: (0, 0)),
            pl.BlockSpec((1, HIDDEN), lambda i: (0, 0)),
            pl.BlockSpec((_VTILE, HIDDEN), lambda i: (i, 0)),
        ],
        out_specs=pl.BlockSpec((NTOK, _VTILE), lambda i: (0, i)),
        out_shape=jax.ShapeDtypeStruct((NTOK, VOCAB), jnp.float32),
        interpret=interpret,
    )(x, w1, b1_2d, wh)


def kernel(input_ids, embed, W1, b1, Wh):
    B, S = input_ids.shape
    ids = input_ids.reshape(NTOK).astype(jnp.int32)
    x = _make_sc_gather()(embed, ids)
    logits = _mlp_tc(x, W1, b1.reshape(1, HIDDEN), Wh)
    return logits.reshape(B, S, VOCAB)


# speedup vs baseline: 1.0054x; 1.0054x over previous
import functools

import jax
import jax.numpy as jnp
from jax import lax
from jax.experimental import pallas as pl
from jax.experimental.pallas import tpu as pltpu
from jax.experimental.pallas import tpu_sc as plsc

VOCAB = 100000
EMBED = 32
HIDDEN = 64
NTOK = 256
CHUNK = 128
PER_CHUNK = CHUNK // EMBED

_VTILE = 4096


@functools.lru_cache(maxsize=1)
def _make_sc_gather():
    info = plsc.get_sparse_core_info()
    nc, ns = info.num_cores, info.num_subcores
    nw = nc * ns
    b_per_w = NTOK // nw
    mesh = plsc.VectorSubcoreMesh(core_axis_name="c", subcore_axis_name="s")

    @functools.partial(
        pl.kernel,
        mesh=mesh,
        out_type=jax.ShapeDtypeStruct((NTOK, CHUNK), jnp.float32),
        scratch_types=[
            pltpu.VMEM((b_per_w,), jnp.int32),
            pltpu.VMEM((b_per_w, CHUNK), jnp.float32),
            pltpu.SemaphoreType.DMA,
        ],
    )
    def gather_kernel(table_hbm, idx_hbm, out_hbm, idx_v, rows_v, sem):
        wid = lax.axis_index("s") * nc + lax.axis_index("c")
        base = wid * b_per_w
        pltpu.sync_copy(idx_hbm.at[pl.ds(base, b_per_w)], idx_v)
        pltpu.async_copy(table_hbm.at[idx_v], rows_v, sem).wait()
        pltpu.sync_copy(rows_v, out_hbm.at[pl.ds(base, b_per_w)])

    return gather_kernel


def _mlp_body(x128_ref, onehot_ref, w1_ref, b1_ref, wh_ref, out_ref, h_ref):
    @pl.when(pl.program_id(0) == 0)
    def _():
        x = jnp.zeros((NTOK, EMBED), jnp.float32)
        for off in range(PER_CHUNK):
            x = x + onehot_ref[:, off:off + 1] * x128_ref[:, off * EMBED:(off + 1) * EMBED]
        h_ref[...] = lax.dot_general(
            x, w1_ref[...],
            (((1,), (1,)), ((), ())),
            preferred_element_type=jnp.float32,
        ) + b1_ref[...]

    out_ref[...] = lax.dot_general(
        h_ref[...], wh_ref[...],
        (((1,), (1,)), ((), ())),
        preferred_element_type=jnp.float32,
    )


def _mlp_tc(x128, onehot, w1, b1_2d, wh, interpret=False):
    grid = (pl.cdiv(VOCAB, _VTILE),)
    return pl.pallas_call(
        _mlp_body,
        grid=grid,
        in_specs=[
            pl.BlockSpec((NTOK, CHUNK), lambda i: (0, 0)),
            pl.BlockSpec((NTOK, PER_CHUNK), lambda i: (0, 0)),
            pl.BlockSpec((HIDDEN, EMBED), lambda i: (0, 0)),
            pl.BlockSpec((1, HIDDEN), lambda i: (0, 0)),
            pl.BlockSpec((_VTILE, HIDDEN), lambda i: (i, 0)),
        ],
        out_specs=pl.BlockSpec((NTOK, _VTILE), lambda i: (0, i)),
        out_shape=jax.ShapeDtypeStruct((NTOK, VOCAB), jnp.float32),
        scratch_shapes=[pltpu.VMEM((NTOK, HIDDEN), jnp.float32)],
        interpret=interpret,
    )(x128, onehot, w1, b1_2d, wh)


def kernel(input_ids, embed, W1, b1, Wh):
    B, S = input_ids.shape
    ids = input_ids.reshape(NTOK).astype(jnp.int32)
    chunk_ids = ids // PER_CHUNK
    onehot = (ids[:, None] % PER_CHUNK == jnp.arange(PER_CHUNK)[None, :]).astype(
        jnp.float32)
    table = embed.reshape(VOCAB // PER_CHUNK, CHUNK)
    x128 = _make_sc_gather()(table, chunk_ids)
    logits = _mlp_tc(x128, onehot, W1, b1.reshape(1, HIDDEN), Wh)
    return logits.reshape(B, S, VOCAB)


# speedup vs baseline: 1.0319x; 1.0264x over previous
import functools

import jax
import jax.numpy as jnp
from jax import lax
from jax.experimental import pallas as pl
from jax.experimental.pallas import tpu as pltpu
from jax.experimental.pallas import tpu_sc as plsc

VOCAB = 100000
EMBED = 32
HIDDEN = 64
NTOK = 256
CHUNK = 128
PER_CHUNK = CHUNK // EMBED

_VTILE = 8192


@functools.lru_cache(maxsize=1)
def _make_sc_gather():
    info = plsc.get_sparse_core_info()
    nc, ns = info.num_cores, info.num_subcores
    nw = nc * ns
    b_per_w = NTOK // nw
    mesh = plsc.VectorSubcoreMesh(core_axis_name="c", subcore_axis_name="s")

    @functools.partial(
        pl.kernel,
        mesh=mesh,
        out_type=jax.ShapeDtypeStruct((NTOK, CHUNK), jnp.float32),
        scratch_types=[
            pltpu.VMEM((b_per_w,), jnp.int32),
            pltpu.VMEM((b_per_w, CHUNK), jnp.float32),
            pltpu.SemaphoreType.DMA,
        ],
    )
    def gather_kernel(table_hbm, idx_hbm, out_hbm, idx_v, rows_v, sem):
        wid = lax.axis_index("s") * nc + lax.axis_index("c")
        base = wid * b_per_w
        pltpu.sync_copy(idx_hbm.at[pl.ds(base, b_per_w)], idx_v)
        pltpu.async_copy(table_hbm.at[idx_v], rows_v, sem).wait()
        pltpu.sync_copy(rows_v, out_hbm.at[pl.ds(base, b_per_w)])

    return gather_kernel


def _mlp_body(x128_ref, onehot_ref, w1_ref, b1_ref, wh_ref, out_ref, h_ref):
    @pl.when(pl.program_id(0) == 0)
    def _():
        x = jnp.zeros((NTOK, EMBED), jnp.float32)
        for off in range(PER_CHUNK):
            x = x + onehot_ref[:, off:off + 1] * x128_ref[:, off * EMBED:(off + 1) * EMBED]
        h_ref[...] = lax.dot_general(
            x, w1_ref[...],
            (((1,), (1,)), ((), ())),
            preferred_element_type=jnp.float32,
        ) + b1_ref[...]

    out_ref[...] = lax.dot_general(
        h_ref[...], wh_ref[...],
        (((1,), (1,)), ((), ())),
        preferred_element_type=jnp.float32,
    )


def _mlp_tc(x128, onehot, w1, b1_2d, wh, interpret=False):
    grid = (pl.cdiv(VOCAB, _VTILE),)
    return pl.pallas_call(
        _mlp_body,
        grid=grid,
        in_specs=[
            pl.BlockSpec((NTOK, CHUNK), lambda i: (0, 0)),
            pl.BlockSpec((NTOK, PER_CHUNK), lambda i: (0, 0)),
            pl.BlockSpec((HIDDEN, EMBED), lambda i: (0, 0)),
            pl.BlockSpec((1, HIDDEN), lambda i: (0, 0)),
            pl.BlockSpec((_VTILE, HIDDEN), lambda i: (i, 0)),
        ],
        out_specs=pl.BlockSpec((NTOK, _VTILE), lambda i: (0, i)),
        out_shape=jax.ShapeDtypeStruct((NTOK, VOCAB), jnp.float32),
        scratch_shapes=[pltpu.VMEM((NTOK, HIDDEN), jnp.float32)],
        interpret=interpret,
    )(x128, onehot, w1, b1_2d, wh)


def kernel(input_ids, embed, W1, b1, Wh):
    B, S = input_ids.shape
    ids = input_ids.reshape(NTOK).astype(jnp.int32)
    chunk_ids = ids // PER_CHUNK
    onehot = (ids[:, None] % PER_CHUNK == jnp.arange(PER_CHUNK)[None, :]).astype(
        jnp.float32)
    table = embed.reshape(VOCAB // PER_CHUNK, CHUNK)
    x128 = _make_sc_gather()(table, chunk_ids)
    logits = _mlp_tc(x128, onehot, W1, b1.reshape(1, HIDDEN), Wh)
    return logits.reshape(B, S, VOCAB)


# speedup vs baseline: 1.3016x; 1.2613x over previous
import jax
import jax.numpy as jnp
from jax import lax
from jax.experimental import pallas as pl
from jax.experimental.pallas import tpu as pltpu

VOCAB = 100000
EMBED = 32
HIDDEN = 64
NTOK = 256

_VTILE = 8192


def _mlp_body(ids_ref, embed_ref, w1_ref, b1_ref, wh_ref, out_ref,
              x_ref, h_ref, sem):
    @pl.when(pl.program_id(0) == 0)
    def _():
        def issue(i, carry):
            r = ids_ref[i]
            pltpu.make_async_copy(
                embed_ref.at[pl.ds(r, 1), :], x_ref.at[pl.ds(i, 1), :], sem
            ).start()
            return carry

        lax.fori_loop(0, NTOK, issue, 0)

        def drain(i, carry):
            pltpu.make_async_copy(
                embed_ref.at[pl.ds(0, 1), :], x_ref.at[pl.ds(i, 1), :], sem
            ).wait()
            return carry

        lax.fori_loop(0, NTOK, drain, 0)

        h_ref[...] = lax.dot_general(
            x_ref[...], w1_ref[...],
            (((1,), (1,)), ((), ())),
            preferred_element_type=jnp.float32,
        ) + b1_ref[...]

    out_ref[...] = lax.dot_general(
        h_ref[...], wh_ref[...],
        (((1,), (1,)), ((), ())),
        preferred_element_type=jnp.float32,
    )


def _mlp_tc(ids, embed, w1, b1_2d, wh, interpret=False):
    grid = (pl.cdiv(VOCAB, _VTILE),)
    return pl.pallas_call(
        _mlp_body,
        grid=grid,
        in_specs=[
            pl.BlockSpec(memory_space=pltpu.MemorySpace.SMEM),
            pl.BlockSpec(memory_space=pltpu.MemorySpace.HBM),
            pl.BlockSpec((HIDDEN, EMBED), lambda i: (0, 0)),
            pl.BlockSpec((1, HIDDEN), lambda i: (0, 0)),
            pl.BlockSpec((_VTILE, HIDDEN), lambda i: (i, 0)),
        ],
        out_specs=pl.BlockSpec((NTOK, _VTILE), lambda i: (0, i)),
        out_shape=jax.ShapeDtypeStruct((NTOK, VOCAB), jnp.float32),
        scratch_shapes=[
            pltpu.VMEM((NTOK, EMBED), jnp.float32),
            pltpu.VMEM((NTOK, HIDDEN), jnp.float32),
            pltpu.SemaphoreType.DMA,
        ],
        interpret=interpret,
    )(ids, embed, w1, b1_2d, wh)


def kernel(input_ids, embed, W1, b1, Wh):
    B, S = input_ids.shape
    ids = input_ids.reshape(NTOK).astype(jnp.int32)
    logits = _mlp_tc(ids, embed, W1, b1.reshape(1, HIDDEN), Wh)
    return logits.reshape(B, S, VOCAB)


# speedup vs baseline: 3.2310x; 2.4824x over previous
import jax
import jax.numpy as jnp
from jax import lax
from jax.experimental import pallas as pl
from jax.experimental.pallas import tpu as pltpu

VOCAB = 100000
EMBED = 32
HIDDEN = 64
NTOK = 256
LANES = 128

_VTILE = 8192


def _mlp_body(ids_ref, oh_ref, embed_t_ref, w1_t_ref, b1_ref, wh_t_ref,
              out_ref, xch_ref, h_ref, sem):
    @pl.when(pl.program_id(0) == 0)
    def _():
        def issue(i, carry):
            r = ids_ref[i]
            c0 = pl.multiple_of((r // LANES) * LANES, LANES)
            pltpu.make_async_copy(
                embed_t_ref.at[:, pl.ds(c0, LANES)], xch_ref.at[i], sem
            ).start()
            return carry

        lax.fori_loop(0, NTOK, issue, 0)

        def drain(i, carry):
            pltpu.make_async_copy(
                embed_t_ref.at[:, pl.ds(0, LANES)], xch_ref.at[i], sem
            ).wait()
            return carry

        lax.fori_loop(0, NTOK, drain, 0)

        x = jnp.sum(xch_ref[...] * oh_ref[...][:, None, :], axis=-1)
        h_ref[...] = lax.dot_general(
            x, w1_t_ref[...],
            (((1,), (0,)), ((), ())),
            preferred_element_type=jnp.float32,
        ) + b1_ref[...]

    out_ref[...] = lax.dot_general(
        h_ref[...], wh_t_ref[...],
        (((1,), (0,)), ((), ())),
        preferred_element_type=jnp.float32,
    )


def _mlp_tc(ids, oh, embed_t, w1_t, b1_2d, wh_t, interpret=False):
    grid = (pl.cdiv(VOCAB, _VTILE),)
    return pl.pallas_call(
        _mlp_body,
        grid=grid,
        in_specs=[
            pl.BlockSpec(memory_space=pltpu.MemorySpace.SMEM),
            pl.BlockSpec((NTOK, LANES), lambda i: (0, 0)),
            pl.BlockSpec(memory_space=pltpu.MemorySpace.HBM),
            pl.BlockSpec((EMBED, HIDDEN), lambda i: (0, 0)),
            pl.BlockSpec((1, HIDDEN), lambda i: (0, 0)),
            pl.BlockSpec((HIDDEN, _VTILE), lambda i: (0, i)),
        ],
        out_specs=pl.BlockSpec((NTOK, _VTILE), lambda i: (0, i)),
        out_shape=jax.ShapeDtypeStruct((NTOK, VOCAB), jnp.float32),
        scratch_shapes=[
            pltpu.VMEM((NTOK, EMBED, LANES), jnp.float32),
            pltpu.VMEM((NTOK, HIDDEN), jnp.float32),
            pltpu.SemaphoreType.DMA,
        ],
        interpret=interpret,
    )(ids, oh, embed_t, w1_t, b1_2d, wh_t)


def kernel(input_ids, embed, W1, b1, Wh):
    B, S = input_ids.shape
    ids = input_ids.reshape(NTOK).astype(jnp.int32)
    oh = (ids[:, None] % LANES == jnp.arange(LANES)[None, :]).astype(jnp.float32)
    logits = _mlp_tc(ids, oh, embed.T, W1.T, b1.reshape(1, HIDDEN), Wh.T)
    return logits.reshape(B, S, VOCAB)


# speedup vs baseline: 3.2909x; 1.0186x over previous
import jax
import jax.numpy as jnp
from jax import lax
from jax.experimental import pallas as pl
from jax.experimental.pallas import tpu as pltpu

VOCAB = 100000
EMBED = 32
HIDDEN = 64
NTOK = 256
LANES = 128

_VTILE = 16384


def _mlp_body(ids_ref, oh_ref, embed_t_ref, w1_t_ref, b1_ref, wh_t_ref,
              out_ref, xch_ref, h_ref, sem):
    @pl.when(pl.program_id(0) == 0)
    def _():
        def issue(i, carry):
            r = ids_ref[i]
            c0 = pl.multiple_of((r // LANES) * LANES, LANES)
            pltpu.make_async_copy(
                embed_t_ref.at[:, pl.ds(c0, LANES)], xch_ref.at[i], sem
            ).start()
            return carry

        lax.fori_loop(0, NTOK, issue, 0)

        def drain(i, carry):
            pltpu.make_async_copy(
                embed_t_ref.at[:, pl.ds(0, LANES)], xch_ref.at[i], sem
            ).wait()
            return carry

        lax.fori_loop(0, NTOK, drain, 0)

        x = jnp.sum(xch_ref[...] * oh_ref[...][:, None, :], axis=-1)
        h_ref[...] = lax.dot_general(
            x, w1_t_ref[...],
            (((1,), (0,)), ((), ())),
            preferred_element_type=jnp.float32,
        ) + b1_ref[...]

    out_ref[...] = lax.dot_general(
        h_ref[...], wh_t_ref[...],
        (((1,), (0,)), ((), ())),
        preferred_element_type=jnp.float32,
    )


def _mlp_tc(ids, oh, embed_t, w1_t, b1_2d, wh_t, interpret=False):
    grid = (pl.cdiv(VOCAB, _VTILE),)
    return pl.pallas_call(
        _mlp_body,
        grid=grid,
        in_specs=[
            pl.BlockSpec(memory_space=pltpu.MemorySpace.SMEM),
            pl.BlockSpec((NTOK, LANES), lambda i: (0, 0)),
            pl.BlockSpec(memory_space=pltpu.MemorySpace.HBM),
            pl.BlockSpec((EMBED, HIDDEN), lambda i: (0, 0)),
            pl.BlockSpec((1, HIDDEN), lambda i: (0, 0)),
            pl.BlockSpec((HIDDEN, _VTILE), lambda i: (0, i)),
        ],
        out_specs=pl.BlockSpec((NTOK, _VTILE), lambda i: (0, i)),
        out_shape=jax.ShapeDtypeStruct((NTOK, VOCAB), jnp.float32),
        scratch_shapes=[
            pltpu.VMEM((NTOK, EMBED, LANES), jnp.float32),
            pltpu.VMEM((NTOK, HIDDEN), jnp.float32),
            pltpu.SemaphoreType.DMA,
        ],
        interpret=interpret,
    )(ids, oh, embed_t, w1_t, b1_2d, wh_t)


def kernel(input_ids, embed, W1, b1, Wh):
    B, S = input_ids.shape
    ids = input_ids.reshape(NTOK).astype(jnp.int32)
    oh = (ids[:, None] % LANES == jnp.arange(LANES)[None, :]).astype(jnp.float32)
    logits = _mlp_tc(ids, oh, embed.T, W1.T, b1.reshape(1, HIDDEN), Wh.T)
    return logits.reshape(B, S, VOCAB)
